# baseline (speedup 1.0000x reference)
import functools

import jax
import jax.numpy as jnp
from jax import lax
from jax.experimental import pallas as pl
from jax.experimental.pallas import tpu as pltpu
from jax.experimental.pallas import tpu_sc as plsc


D = 128
CHUNK = 128
NBUF = 7


def _gather_kernel(B, n_workers):
    b_per_w = B // n_workers
    n_chunks = b_per_w // CHUNK
    assert b_per_w % CHUNK == 0 and n_chunks > 2 * NBUF

    mesh = plsc.VectorSubcoreMesh(core_axis_name="c", subcore_axis_name="s")

    @functools.partial(
        pl.kernel,
        mesh=mesh,
        out_type=jax.ShapeDtypeStruct((B, D), jnp.float32),
        scratch_types=[
            pltpu.VMEM((b_per_w,), jnp.int32),
            pltpu.VMEM((NBUF, CHUNK, D), jnp.float32),
            pltpu.SemaphoreType.DMA((NBUF,)),
            pltpu.SemaphoreType.DMA((NBUF,)),
        ],
    )
    def body(idx_hbm, table_hbm, out_hbm, idx_v, rows_v, gsem, ssem):
        n_cores = lax.axis_size("c")
        wid = lax.axis_index("s") * n_cores + lax.axis_index("c")
        base = wid * b_per_w

        pltpu.sync_copy(idx_hbm.at[wid], idx_v)

        def start_gather(j, b):
            pltpu.async_copy(table_hbm.at[idx_v.at[pl.ds(j * CHUNK, CHUNK)]],
                             rows_v.at[b], gsem.at[b])

        def wait_gather(b):
            pltpu.make_async_copy(table_hbm.at[pl.ds(0, CHUNK)],
                                  rows_v.at[b], gsem.at[b]).wait()

        def start_scatter(j, b):
            pltpu.async_copy(rows_v.at[b],
                             out_hbm.at[pl.ds(base + j * CHUNK, CHUNK)],
                             ssem.at[b])

        def wait_scatter(b):
            pltpu.make_async_copy(rows_v.at[b],
                                  out_hbm.at[pl.ds(base, CHUNK)],
                                  ssem.at[b]).wait()

        for b in range(NBUF):
            start_gather(b, b)

        wait_gather(0)
        start_scatter(0, 0)

        def step(j):
            wait_scatter((j - 1) % NBUF)
            start_gather(j - 1 + NBUF, (j - 1) % NBUF)
            wait_gather(j % NBUF)
            start_scatter(j, j % NBUF)

        n_grouped = ((n_chunks - NBUF - 1) // NBUF) * NBUF

        @pl.loop(0, n_grouped, step=NBUF)
        def _(k):
            for i in range(NBUF):
                j = k + i + 1
                wait_scatter(i % NBUF)
                start_gather(j - 1 + NBUF, i % NBUF)
                wait_gather((i + 1) % NBUF)
                start_scatter(j, (i + 1) % NBUF)

        for j in range(n_grouped + 1, n_chunks - NBUF + 1):
            step(j)
        for j in range(n_chunks - NBUF + 1, n_chunks):
            wait_gather(j % NBUF)
            start_scatter(j, j % NBUF)
        for b in range(NBUF):
            wait_scatter(b)

    return body


def _impl(data, ivectors):
    n_rows, n_cols = data.shape
    B = n_rows * n_cols
    info = plsc.get_sparse_core_info()
    n_workers = info.num_cores * info.num_subcores
    idx = data.astype(jnp.int32).T.reshape(n_workers, B // n_workers)
    out = _gather_kernel(B, n_workers)(idx, ivectors)
    return out.reshape(n_cols, n_rows, D).transpose(1, 0, 2)


kernel = jax.jit(_impl)


# speedup vs baseline: 1.0176x; 1.0176x over previous
import functools

import jax
import jax.numpy as jnp
from jax import lax
from jax.experimental import pallas as pl
from jax.experimental.pallas import tpu as pltpu
from jax.experimental.pallas import tpu_sc as plsc


D = 128
CHUNK = 128
NBUF = 7


def _gather_kernel(n_rows, n_cols, n_workers):
    B = n_rows * n_cols
    n_chunks = n_cols
    assert n_rows % (n_workers * CHUNK) == 0 or n_rows == n_workers * CHUNK
    assert n_chunks > 2 * NBUF

    mesh = plsc.VectorSubcoreMesh(core_axis_name="c", subcore_axis_name="s")

    @functools.partial(
        pl.kernel,
        mesh=mesh,
        out_type=jax.ShapeDtypeStruct((B, D), jnp.float32),
        scratch_types=[
            pltpu.VMEM((n_chunks, CHUNK), jnp.int32),
            pltpu.VMEM((NBUF, CHUNK, D), jnp.float32),
            pltpu.SemaphoreType.DMA((NBUF,)),
            pltpu.SemaphoreType.DMA((NBUF,)),
        ],
    )
    def body(idx_hbm, table_hbm, out_hbm, idx_v, rows_v, gsem, ssem):
        n_cores = lax.axis_size("c")
        wid = lax.axis_index("s") * n_cores + lax.axis_index("c")
        col0 = wid * CHUNK

        pltpu.sync_copy(idx_hbm.at[:, pl.ds(col0, CHUNK)], idx_v)

        def start_gather(j, b):
            pltpu.async_copy(table_hbm.at[idx_v.at[j]], rows_v.at[b],
                             gsem.at[b])

        def wait_gather(b):
            pltpu.make_async_copy(table_hbm.at[pl.ds(0, CHUNK)],
                                  rows_v.at[b], gsem.at[b]).wait()

        def start_scatter(j, b):
            pltpu.async_copy(rows_v.at[b],
                             out_hbm.at[pl.ds(j * n_rows + col0, CHUNK)],
                             ssem.at[b])

        def wait_scatter(b):
            pltpu.make_async_copy(rows_v.at[b],
                                  out_hbm.at[pl.ds(col0, CHUNK)],
                                  ssem.at[b]).wait()

        for b in range(NBUF):
            start_gather(b, b)

        wait_gather(0)
        start_scatter(0, 0)

        def step(j):
            wait_scatter((j - 1) % NBUF)
            start_gather(j - 1 + NBUF, (j - 1) % NBUF)
            wait_gather(j % NBUF)
            start_scatter(j, j % NBUF)

        n_grouped = ((n_chunks - NBUF - 1) // NBUF) * NBUF

        @pl.loop(0, n_grouped, step=NBUF)
        def _(k):
            for i in range(NBUF):
                j = k + i + 1
                wait_scatter(i % NBUF)
                start_gather(j - 1 + NBUF, i % NBUF)
                wait_gather((i + 1) % NBUF)
                start_scatter(j, (i + 1) % NBUF)

        for j in range(n_grouped + 1, n_chunks - NBUF + 1):
            step(j)
        for j in range(n_chunks - NBUF + 1, n_chunks):
            wait_gather(j % NBUF)
            start_scatter(j, j % NBUF)
        for b in range(NBUF):
            wait_scatter(b)

    return body


def _impl(data, ivectors):
    n_rows, n_cols = data.shape
    info = plsc.get_sparse_core_info()
    n_workers = info.num_cores * info.num_subcores
    idx = data.astype(jnp.int32).T
    out = _gather_kernel(n_rows, n_cols, n_workers)(idx, ivectors)
    return out.reshape(n_cols, n_rows, D).transpose(1, 0, 2)


kernel = jax.jit(_impl)


# speedup vs baseline: 1.0181x; 1.0005x over previous
import functools

import jax
import jax.numpy as jnp
from jax import lax
from jax.experimental import pallas as pl
from jax.experimental.pallas import tpu as pltpu
from jax.experimental.pallas import tpu_sc as plsc


D = 128
CHUNK = 128
NBUF = 7


def _gather_kernel(n_rows, n_cols, n_workers):
    B = n_rows * n_cols
    n_chunks = n_cols
    assert n_rows % (n_workers * CHUNK) == 0 or n_rows == n_workers * CHUNK
    assert n_chunks > 2 * NBUF

    mesh = plsc.VectorSubcoreMesh(core_axis_name="c", subcore_axis_name="s")

    @functools.partial(
        pl.kernel,
        mesh=mesh,
        out_type=jax.ShapeDtypeStruct((B, D), jnp.float32),
        scratch_types=[
            pltpu.VMEM((n_chunks, CHUNK), jnp.int32),
            pltpu.VMEM((NBUF, CHUNK, D), jnp.float32),
            pltpu.SemaphoreType.DMA((NBUF,)),
            pltpu.SemaphoreType.DMA((NBUF,)),
            pltpu.SemaphoreType.DMA,
        ],
    )
    def body(idx_hbm, table_hbm, out_hbm, idx_v, rows_v, gsem, ssem, isem):
        n_cores = lax.axis_size("c")
        wid = lax.axis_index("s") * n_cores + lax.axis_index("c")
        col0 = wid * CHUNK

        pltpu.sync_copy(idx_hbm.at[pl.ds(0, 8), pl.ds(col0, CHUNK)],
                        idx_v.at[pl.ds(0, 8)])

        def start_gather(j, b):
            pltpu.async_copy(table_hbm.at[idx_v.at[j]], rows_v.at[b],
                             gsem.at[b])

        def wait_gather(b):
            pltpu.make_async_copy(table_hbm.at[pl.ds(0, CHUNK)],
                                  rows_v.at[b], gsem.at[b]).wait()

        def start_scatter(j, b):
            pltpu.async_copy(rows_v.at[b],
                             out_hbm.at[pl.ds(j * n_rows + col0, CHUNK)],
                             ssem.at[b])

        def wait_scatter(b):
            pltpu.make_async_copy(rows_v.at[b],
                                  out_hbm.at[pl.ds(col0, CHUNK)],
                                  ssem.at[b]).wait()

        for b in range(NBUF):
            start_gather(b, b)

        idx_rest = pltpu.async_copy(
            idx_hbm.at[pl.ds(8, n_chunks - 8), pl.ds(col0, CHUNK)],
            idx_v.at[pl.ds(8, n_chunks - 8)], isem)

        wait_gather(0)
        start_scatter(0, 0)
        idx_rest.wait()

        def step(j):
            wait_scatter((j - 1) % NBUF)
            start_gather(j - 1 + NBUF, (j - 1) % NBUF)
            wait_gather(j % NBUF)
            start_scatter(j, j % NBUF)

        n_grouped = ((n_chunks - NBUF - 1) // NBUF) * NBUF

        @pl.loop(0, n_grouped, step=NBUF)
        def _(k):
            for i in range(NBUF):
                j = k + i + 1
                wait_scatter(i % NBUF)
                start_gather(j - 1 + NBUF, i % NBUF)
                wait_gather((i + 1) % NBUF)
                start_scatter(j, (i + 1) % NBUF)

        for j in range(n_grouped + 1, n_chunks - NBUF + 1):
            step(j)
        for j in range(n_chunks - NBUF + 1, n_chunks):
            wait_gather(j % NBUF)
            start_scatter(j, j % NBUF)
        for b in range(NBUF):
            wait_scatter(b)

    return body


def _impl(data, ivectors):
    n_rows, n_cols = data.shape
    info = plsc.get_sparse_core_info()
    n_workers = info.num_cores * info.num_subcores
    idx = data.astype(jnp.int32).T
    out = _gather_kernel(n_rows, n_cols, n_workers)(idx, ivectors)
    return out.reshape(n_cols, n_rows, D).transpose(1, 0, 2)


kernel = jax.jit(_impl)
